# baseline (speedup 1.0000x reference)
import jax
import jax.numpy as jnp
from jax.experimental import pallas as pl
from jax.experimental.pallas import tpu as pltpu
from jax.experimental.pallas import tpu_sc as plsc

NC = 2
NS = 16
LANES = 16

NUM_EXPERTS = 8
TOP_K = 2
HIDDEN = 1024
INTER = 1024
TOKENS = 2048

BM = 1024
NT = (TOP_K * TOKENS) // BM + NUM_EXPERTS
R = NT * BM
P = TOP_K * TOKENS

NW = NC * NS
W_TOK = TOKENS // NW
CHUNK = 32
N_CHUNKS = W_TOK // CHUNK


def _routing_kernel(x_ref, gw_ref, pos_ref, pwb_ref, te_ref, nv_ref):
    logits = jax.lax.dot_general(
        x_ref[:], gw_ref[:], (((1,), (1,)), ((), ())),
        preferred_element_type=jnp.float32)
    T = logits.shape[0]
    e_iota = jax.lax.broadcasted_iota(jnp.int32, logits.shape, 1)
    m1 = jnp.max(logits, axis=1, keepdims=True)
    idx1 = jnp.min(jnp.where(logits == m1, e_iota, NUM_EXPERTS),
                   axis=1, keepdims=True)
    oh1 = e_iota == idx1
    l2 = jnp.where(oh1, -jnp.inf, logits)
    m2 = jnp.max(l2, axis=1, keepdims=True)
    idx2 = jnp.min(jnp.where(l2 == m2, e_iota, NUM_EXPERTS),
                   axis=1, keepdims=True)
    oh2 = e_iota == idx2
    t = jnp.exp(m2 - m1)
    p1 = 1.0 / (1.0 + t)
    p2 = t / (1.0 + t)

    oh1f = oh1.astype(jnp.float32)
    oh2f = oh2.astype(jnp.float32)
    ohs = oh1f + oh2f

    NB, BT = 16, 128
    ohs3 = ohs.reshape(NB, BT, NUM_EXPERTS)
    rb_i = jax.lax.broadcasted_iota(jnp.int32, (BT, BT), 0)
    cb_i = jax.lax.broadcasted_iota(jnp.int32, (BT, BT), 1)
    ltb = jnp.broadcast_to((rb_i > cb_i).astype(jnp.float32), (NB, BT, BT))
    within = jax.lax.dot_general(ltb, ohs3, (((2,), (1,)), ((0,), (0,))),
                                 preferred_element_type=jnp.float32)
    bs = jnp.sum(ohs3, axis=1)
    rn_i = jax.lax.broadcasted_iota(jnp.int32, (NB, NB), 0)
    cn_i = jax.lax.broadcasted_iota(jnp.int32, (NB, NB), 1)
    ltn = (rn_i > cn_i).astype(jnp.float32)
    bp = jax.lax.dot_general(ltn, bs, (((1,), (0,)), ((), ())),
                             preferred_element_type=jnp.float32)
    cc = (within + bp[:, None, :]).reshape(T, NUM_EXPERTS)

    ones_col = jnp.ones((T, 1), jnp.float32)
    counts_col = jax.lax.dot_general(ohs, ones_col, (((0,), (0,)), ((), ())),
                                     preferred_element_type=jnp.float32)
    tiles_col = jnp.floor((counts_col + float(BM - 1)) * (1.0 / BM))
    re_i = jax.lax.broadcasted_iota(jnp.int32, (NUM_EXPERTS, NUM_EXPERTS), 0)
    ce_i = jax.lax.broadcasted_iota(jnp.int32, (NUM_EXPERTS, NUM_EXPERTS), 1)
    l8_strict = (re_i > ce_i).astype(jnp.float32)
    l8 = (re_i >= ce_i).astype(jnp.float32)
    ts_excl_col = jax.lax.dot_general(
        l8_strict, tiles_col, (((1,), (0,)), ((), ())),
        preferred_element_type=jnp.float32)
    ts_incl_col = jax.lax.dot_general(
        l8, tiles_col, (((1,), (0,)), ((), ())),
        preferred_element_type=jnp.float32)
    ps_col = ts_excl_col * float(BM)

    pos1 = jax.lax.dot_general(oh1f, ps_col, (((1,), (0,)), ((), ())),
                               preferred_element_type=jnp.float32)
    pos1 = pos1 + jnp.sum(oh1f * cc, axis=1, keepdims=True)
    pos2 = jax.lax.dot_general(oh2f, ps_col, (((1,), (0,)), ((), ())),
                               preferred_element_type=jnp.float32)
    pos2 = pos2 + jnp.sum(oh2f * cc, axis=1, keepdims=True)
    pos_ref[:] = jnp.round(
        jnp.concatenate([pos1, pos2], axis=1)).astype(jnp.int32)

    pwb_ref[:] = jnp.concatenate([p1, p2], axis=1)

    total = ts_incl_col[NUM_EXPERTS - 1:NUM_EXPERTS, :]
    i_row = jax.lax.broadcasted_iota(jnp.int32, (1, NT), 1).astype(jnp.float32)
    i_eff = jnp.minimum(i_row, total - 1.0)
    in_e = jnp.logical_and(ts_excl_col <= i_eff, i_eff < ts_incl_col)
    e_row = jax.lax.broadcasted_iota(
        jnp.int32, (1, NUM_EXPERTS), 1).astype(jnp.float32)
    te_ref[:] = jnp.round(jax.lax.dot_general(
        e_row, jnp.where(in_e, 1.0, 0.0), (((1,), (0,)), ((), ())),
        preferred_element_type=jnp.float32)).astype(jnp.int32)
    nv_ref[:] = jnp.round(total).astype(jnp.int32)


_SC_MESH = plsc.VectorSubcoreMesh(
    core_axis_name="c", subcore_axis_name="s", num_cores=NC, num_subcores=NS)


def _dispatch_body(x_hbm, posd_hbm, xs_hbm, idx_v, xb_v, lsem, ssem):
    c = jax.lax.axis_index("c")
    s = jax.lax.axis_index("s")
    wid = c * NS + s
    tok0 = wid * W_TOK
    l0 = pltpu.async_copy(posd_hbm.at[wid], idx_v, lsem)
    l1 = pltpu.async_copy(x_hbm.at[pl.ds(tok0, W_TOK)], xb_v, lsem)
    l0.wait()
    l1.wait()
    s0 = pltpu.async_copy(xb_v, xs_hbm.at[idx_v.at[0]], ssem)
    s1 = pltpu.async_copy(xb_v, xs_hbm.at[idx_v.at[1]], ssem)
    s0.wait()
    s1.wait()


_dispatch_call = pl.kernel(
    _dispatch_body,
    out_type=jax.ShapeDtypeStruct((R, HIDDEN), jnp.float32),
    mesh=_SC_MESH,
    scratch_types=[
        pltpu.VMEM((TOP_K, W_TOK), jnp.int32),
        pltpu.VMEM((W_TOK, HIDDEN), jnp.float32),
        pltpu.SemaphoreType.DMA,
        pltpu.SemaphoreType.DMA,
    ],
)


def _combine_body(y_hbm, posd_hbm, ya_hbm, yb_hbm,
                  idx_v, b0, b1, b2, lsem, gsem, osem):
    c = jax.lax.axis_index("c")
    s = jax.lax.axis_index("s")
    wid = c * NS + s
    tok0 = wid * W_TOK
    pltpu.sync_copy(posd_hbm.at[wid], idx_v)
    g0 = pltpu.async_copy(y_hbm.at[idx_v.at[0, 0]], b0, gsem)
    g1 = pltpu.async_copy(y_hbm.at[idx_v.at[0, 1]], b1, gsem)
    g2 = pltpu.async_copy(y_hbm.at[idx_v.at[1, 0]], b2, gsem)
    g0.wait()
    w0 = pltpu.async_copy(b0, ya_hbm.at[pl.ds(tok0, CHUNK)], osem)
    g1.wait()
    w1 = pltpu.async_copy(b1, yb_hbm.at[pl.ds(tok0, CHUNK)], osem)
    w0.wait()
    g3 = pltpu.async_copy(y_hbm.at[idx_v.at[1, 1]], b0, gsem)
    g2.wait()
    w2 = pltpu.async_copy(b2, ya_hbm.at[pl.ds(tok0 + CHUNK, CHUNK)], osem)
    g3.wait()
    w3 = pltpu.async_copy(b0, yb_hbm.at[pl.ds(tok0 + CHUNK, CHUNK)], osem)
    w1.wait()
    w2.wait()
    w3.wait()


_combine_call = pl.kernel(
    _combine_body,
    out_type=(
        jax.ShapeDtypeStruct((TOKENS, HIDDEN), jnp.float32),
        jax.ShapeDtypeStruct((TOKENS, HIDDEN), jnp.float32),
    ),
    mesh=_SC_MESH,
    scratch_types=[
        pltpu.VMEM((N_CHUNKS, 2, CHUNK), jnp.int32),
        pltpu.VMEM((CHUNK, HIDDEN), jnp.float32),
        pltpu.VMEM((CHUNK, HIDDEN), jnp.float32),
        pltpu.VMEM((CHUNK, HIDDEN), jnp.float32),
        pltpu.SemaphoreType.DMA,
        pltpu.SemaphoreType.DMA,
        pltpu.SemaphoreType.DMA,
    ],
)


def _add_kernel(a_ref, b_ref, w_ref, o_ref):
    w = w_ref[:]
    o_ref[:] = a_ref[:] * w[:, 0:1] + b_ref[:] * w[:, 1:2]


def _ffn_kernel(te_ref, nv_ref, xs_ref, fc_ref, proj_ref, y_ref):
    i = pl.program_id(0)

    @pl.when(i < nv_ref[0, 0])
    def _():
        a = xs_ref[:]
        wfc = fc_ref[0]
        u = jax.lax.dot_general(a, wfc[:INTER], (((1,), (1,)), ((), ())),
                                preferred_element_type=jnp.float32)
        g = jax.lax.dot_general(a, wfc[INTER:], (((1,), (1,)), ((), ())),
                                preferred_element_type=jnp.float32)
        h = u * (g * jax.nn.sigmoid(g))
        y_ref[:] = jax.lax.dot_general(
            h, proj_ref[0], (((1,), (1,)), ((), ())),
            preferred_element_type=jnp.float32)


@jax.jit
def kernel(hidden_states, gate_w, c_fc_w, c_proj_w):
    T, H = hidden_states.shape

    pos_tm, pwb, te, nv = pl.pallas_call(
        _routing_kernel,
        out_shape=(
            jax.ShapeDtypeStruct((T, TOP_K), jnp.int32),
            jax.ShapeDtypeStruct((T, TOP_K), jnp.float32),
            jax.ShapeDtypeStruct((1, NT), jnp.int32),
            jax.ShapeDtypeStruct((1, 1), jnp.int32),
        ),
    )(hidden_states, gate_w)

    posd = pos_tm.reshape(NW, N_CHUNKS, CHUNK, TOP_K).transpose(0, 1, 3, 2)
    posw = pos_tm.reshape(NW, W_TOK, TOP_K).transpose(0, 2, 1)

    xs = _dispatch_call(hidden_states, posw)

    y_rows = pl.pallas_call(
        _ffn_kernel,
        grid_spec=pltpu.PrefetchScalarGridSpec(
            num_scalar_prefetch=2,
            grid=(NT,),
            in_specs=[
                pl.BlockSpec((BM, H),
                             lambda i, te, nv: (jnp.minimum(i, nv[0, 0] - 1), 0)),
                pl.BlockSpec((1, 2 * INTER, H),
                             lambda i, te, nv: (te[0, i], 0, 0)),
                pl.BlockSpec((1, H, INTER),
                             lambda i, te, nv: (te[0, i], 0, 0)),
            ],
            out_specs=pl.BlockSpec(
                (BM, H), lambda i, te, nv: (jnp.minimum(i, nv[0, 0] - 1), 0)),
        ),
        out_shape=jax.ShapeDtypeStruct((R, H), jnp.float32),
    )(te, nv, xs, c_fc_w, c_proj_w)

    ya, yb = _combine_call(y_rows, posd)
    out = pl.pallas_call(
        _add_kernel,
        grid=(4,),
        in_specs=[
            pl.BlockSpec((T // 4, H), lambda i: (i, 0)),
            pl.BlockSpec((T // 4, H), lambda i: (i, 0)),
            pl.BlockSpec((T // 4, TOP_K), lambda i: (i, 0)),
        ],
        out_specs=pl.BlockSpec((T // 4, H), lambda i: (i, 0)),
        out_shape=jax.ShapeDtypeStruct((T, H), jnp.float32),
    )(ya, yb, pwb)
    return out


# speedup vs baseline: 1.0772x; 1.0772x over previous
import jax
import jax.numpy as jnp
from jax.experimental import pallas as pl
from jax.experimental.pallas import tpu as pltpu
from jax.experimental.pallas import tpu_sc as plsc

NC = 2
NS = 16
LANES = 16

NUM_EXPERTS = 8
TOP_K = 2
HIDDEN = 1024
INTER = 1024
TOKENS = 2048

BM = 512
NT = (TOP_K * TOKENS) // BM + NUM_EXPERTS
R = NT * BM
P = TOP_K * TOKENS

NW = NC * NS
W_TOK = TOKENS // NW
CHUNK = 32
N_CHUNKS = W_TOK // CHUNK


def _routing_kernel(x_ref, gw_ref, pos_ref, pwb_ref, te_ref, nv_ref):
    logits = jax.lax.dot_general(
        x_ref[:], gw_ref[:], (((1,), (1,)), ((), ())),
        preferred_element_type=jnp.float32)
    T = logits.shape[0]
    e_iota = jax.lax.broadcasted_iota(jnp.int32, logits.shape, 1)
    m1 = jnp.max(logits, axis=1, keepdims=True)
    idx1 = jnp.min(jnp.where(logits == m1, e_iota, NUM_EXPERTS),
                   axis=1, keepdims=True)
    oh1 = e_iota == idx1
    l2 = jnp.where(oh1, -jnp.inf, logits)
    m2 = jnp.max(l2, axis=1, keepdims=True)
    idx2 = jnp.min(jnp.where(l2 == m2, e_iota, NUM_EXPERTS),
                   axis=1, keepdims=True)
    oh2 = e_iota == idx2
    t = jnp.exp(m2 - m1)
    p1 = 1.0 / (1.0 + t)
    p2 = t / (1.0 + t)

    oh1f = oh1.astype(jnp.float32)
    oh2f = oh2.astype(jnp.float32)
    ohs = oh1f + oh2f

    NB, BT = 16, 128
    ohs3 = ohs.reshape(NB, BT, NUM_EXPERTS)
    rb_i = jax.lax.broadcasted_iota(jnp.int32, (BT, BT), 0)
    cb_i = jax.lax.broadcasted_iota(jnp.int32, (BT, BT), 1)
    ltb = jnp.broadcast_to((rb_i > cb_i).astype(jnp.float32), (NB, BT, BT))
    within = jax.lax.dot_general(ltb, ohs3, (((2,), (1,)), ((0,), (0,))),
                                 preferred_element_type=jnp.float32)
    bs = jnp.sum(ohs3, axis=1)
    rn_i = jax.lax.broadcasted_iota(jnp.int32, (NB, NB), 0)
    cn_i = jax.lax.broadcasted_iota(jnp.int32, (NB, NB), 1)
    ltn = (rn_i > cn_i).astype(jnp.float32)
    bp = jax.lax.dot_general(ltn, bs, (((1,), (0,)), ((), ())),
                             preferred_element_type=jnp.float32)
    cc = (within + bp[:, None, :]).reshape(T, NUM_EXPERTS)

    ones_col = jnp.ones((T, 1), jnp.float32)
    counts_col = jax.lax.dot_general(ohs, ones_col, (((0,), (0,)), ((), ())),
                                     preferred_element_type=jnp.float32)
    tiles_col = jnp.floor((counts_col + float(BM - 1)) * (1.0 / BM))
    re_i = jax.lax.broadcasted_iota(jnp.int32, (NUM_EXPERTS, NUM_EXPERTS), 0)
    ce_i = jax.lax.broadcasted_iota(jnp.int32, (NUM_EXPERTS, NUM_EXPERTS), 1)
    l8_strict = (re_i > ce_i).astype(jnp.float32)
    l8 = (re_i >= ce_i).astype(jnp.float32)
    ts_excl_col = jax.lax.dot_general(
        l8_strict, tiles_col, (((1,), (0,)), ((), ())),
        preferred_element_type=jnp.float32)
    ts_incl_col = jax.lax.dot_general(
        l8, tiles_col, (((1,), (0,)), ((), ())),
        preferred_element_type=jnp.float32)
    ps_col = ts_excl_col * float(BM)

    pos1 = jax.lax.dot_general(oh1f, ps_col, (((1,), (0,)), ((), ())),
                               preferred_element_type=jnp.float32)
    pos1 = pos1 + jnp.sum(oh1f * cc, axis=1, keepdims=True)
    pos2 = jax.lax.dot_general(oh2f, ps_col, (((1,), (0,)), ((), ())),
                               preferred_element_type=jnp.float32)
    pos2 = pos2 + jnp.sum(oh2f * cc, axis=1, keepdims=True)
    pos_ref[:] = jnp.round(
        jnp.concatenate([pos1, pos2], axis=1)).astype(jnp.int32)

    pwb_ref[:] = jnp.concatenate([p1, p2], axis=1)

    total = ts_incl_col[NUM_EXPERTS - 1:NUM_EXPERTS, :]
    i_row = jax.lax.broadcasted_iota(jnp.int32, (1, NT), 1).astype(jnp.float32)
    i_eff = jnp.minimum(i_row, total - 1.0)
    in_e = jnp.logical_and(ts_excl_col <= i_eff, i_eff < ts_incl_col)
    e_row = jax.lax.broadcasted_iota(
        jnp.int32, (1, NUM_EXPERTS), 1).astype(jnp.float32)
    te_ref[:] = jnp.round(jax.lax.dot_general(
        e_row, jnp.where(in_e, 1.0, 0.0), (((1,), (0,)), ((), ())),
        preferred_element_type=jnp.float32)).astype(jnp.int32)
    nv_ref[:] = jnp.round(total).astype(jnp.int32)


_SC_MESH = plsc.VectorSubcoreMesh(
    core_axis_name="c", subcore_axis_name="s", num_cores=NC, num_subcores=NS)


def _dispatch_body(x_hbm, posd_hbm, xs_hbm, idx_v, xb_v, lsem, ssem):
    c = jax.lax.axis_index("c")
    s = jax.lax.axis_index("s")
    wid = c * NS + s
    tok0 = wid * W_TOK
    l0 = pltpu.async_copy(posd_hbm.at[wid], idx_v, lsem)
    l1 = pltpu.async_copy(x_hbm.at[pl.ds(tok0, W_TOK)], xb_v, lsem)
    l0.wait()
    l1.wait()
    s0 = pltpu.async_copy(xb_v, xs_hbm.at[idx_v.at[0]], ssem)
    s1 = pltpu.async_copy(xb_v, xs_hbm.at[idx_v.at[1]], ssem)
    s0.wait()
    s1.wait()


_dispatch_call = pl.kernel(
    _dispatch_body,
    out_type=jax.ShapeDtypeStruct((R, HIDDEN), jnp.float32),
    mesh=_SC_MESH,
    scratch_types=[
        pltpu.VMEM((TOP_K, W_TOK), jnp.int32),
        pltpu.VMEM((W_TOK, HIDDEN), jnp.float32),
        pltpu.SemaphoreType.DMA,
        pltpu.SemaphoreType.DMA,
    ],
)


def _combine_body(y_hbm, posd_hbm, ya_hbm, yb_hbm,
                  idx_v, b0, b1, b2, lsem, gsem, osem):
    c = jax.lax.axis_index("c")
    s = jax.lax.axis_index("s")
    wid = c * NS + s
    tok0 = wid * W_TOK
    pltpu.sync_copy(posd_hbm.at[wid], idx_v)
    g0 = pltpu.async_copy(y_hbm.at[idx_v.at[0, 0]], b0, gsem)
    g1 = pltpu.async_copy(y_hbm.at[idx_v.at[0, 1]], b1, gsem)
    g2 = pltpu.async_copy(y_hbm.at[idx_v.at[1, 0]], b2, gsem)
    g0.wait()
    w0 = pltpu.async_copy(b0, ya_hbm.at[pl.ds(tok0, CHUNK)], osem)
    g1.wait()
    w1 = pltpu.async_copy(b1, yb_hbm.at[pl.ds(tok0, CHUNK)], osem)
    w0.wait()
    g3 = pltpu.async_copy(y_hbm.at[idx_v.at[1, 1]], b0, gsem)
    g2.wait()
    w2 = pltpu.async_copy(b2, ya_hbm.at[pl.ds(tok0 + CHUNK, CHUNK)], osem)
    g3.wait()
    w3 = pltpu.async_copy(b0, yb_hbm.at[pl.ds(tok0 + CHUNK, CHUNK)], osem)
    w1.wait()
    w2.wait()
    w3.wait()


_combine_call = pl.kernel(
    _combine_body,
    out_type=(
        jax.ShapeDtypeStruct((TOKENS, HIDDEN), jnp.float32),
        jax.ShapeDtypeStruct((TOKENS, HIDDEN), jnp.float32),
    ),
    mesh=_SC_MESH,
    scratch_types=[
        pltpu.VMEM((N_CHUNKS, 2, CHUNK), jnp.int32),
        pltpu.VMEM((CHUNK, HIDDEN), jnp.float32),
        pltpu.VMEM((CHUNK, HIDDEN), jnp.float32),
        pltpu.VMEM((CHUNK, HIDDEN), jnp.float32),
        pltpu.SemaphoreType.DMA,
        pltpu.SemaphoreType.DMA,
        pltpu.SemaphoreType.DMA,
    ],
)


def _add_kernel(a_ref, b_ref, w_ref, o_ref):
    w = w_ref[:]
    o_ref[:] = a_ref[:] * w[:, 0:1] + b_ref[:] * w[:, 1:2]


def _ffn_kernel(te_ref, nv_ref, xs_ref, fc_ref, proj_ref, y_ref):
    i = pl.program_id(0)

    @pl.when(i < nv_ref[0, 0])
    def _():
        a = xs_ref[:]
        wfc = fc_ref[0]
        u = jax.lax.dot_general(a, wfc[:INTER], (((1,), (1,)), ((), ())),
                                preferred_element_type=jnp.float32)
        g = jax.lax.dot_general(a, wfc[INTER:], (((1,), (1,)), ((), ())),
                                preferred_element_type=jnp.float32)
        h = u * (g * jax.nn.sigmoid(g))
        y_ref[:] = jax.lax.dot_general(
            h, proj_ref[0], (((1,), (1,)), ((), ())),
            preferred_element_type=jnp.float32)


@jax.jit
def kernel(hidden_states, gate_w, c_fc_w, c_proj_w):
    T, H = hidden_states.shape

    pos_tm, pwb, te, nv = pl.pallas_call(
        _routing_kernel,
        out_shape=(
            jax.ShapeDtypeStruct((T, TOP_K), jnp.int32),
            jax.ShapeDtypeStruct((T, TOP_K), jnp.float32),
            jax.ShapeDtypeStruct((1, NT), jnp.int32),
            jax.ShapeDtypeStruct((1, 1), jnp.int32),
        ),
    )(hidden_states, gate_w)

    posd = pos_tm.reshape(NW, N_CHUNKS, CHUNK, TOP_K).transpose(0, 1, 3, 2)
    posw = pos_tm.reshape(NW, W_TOK, TOP_K).transpose(0, 2, 1)

    xs = _dispatch_call(hidden_states, posw)

    y_rows = pl.pallas_call(
        _ffn_kernel,
        grid_spec=pltpu.PrefetchScalarGridSpec(
            num_scalar_prefetch=2,
            grid=(NT,),
            in_specs=[
                pl.BlockSpec((BM, H),
                             lambda i, te, nv: (jnp.minimum(i, nv[0, 0] - 1), 0)),
                pl.BlockSpec((1, 2 * INTER, H),
                             lambda i, te, nv: (te[0, i], 0, 0)),
                pl.BlockSpec((1, H, INTER),
                             lambda i, te, nv: (te[0, i], 0, 0)),
            ],
            out_specs=pl.BlockSpec(
                (BM, H), lambda i, te, nv: (jnp.minimum(i, nv[0, 0] - 1), 0)),
        ),
        out_shape=jax.ShapeDtypeStruct((R, H), jnp.float32),
    )(te, nv, xs, c_fc_w, c_proj_w)

    ya, yb = _combine_call(y_rows, posd)
    out = pl.pallas_call(
        _add_kernel,
        grid=(4,),
        in_specs=[
            pl.BlockSpec((T // 4, H), lambda i: (i, 0)),
            pl.BlockSpec((T // 4, H), lambda i: (i, 0)),
            pl.BlockSpec((T // 4, TOP_K), lambda i: (i, 0)),
        ],
        out_specs=pl.BlockSpec((T // 4, H), lambda i: (i, 0)),
        out_shape=jax.ShapeDtypeStruct((T, H), jnp.float32),
    )(ya, yb, pwb)
    return out
